# baseline (speedup 1.0000x reference)
import functools

import jax
import jax.numpy as jnp
from jax import lax
from jax.experimental import pallas as pl
from jax.experimental.pallas import tpu as pltpu
from jax.experimental.pallas import tpu_sc as plsc

VOCAB = 100000
EMB = 128
CS = 64
CTX = 50
NEG = 10

ROWS = 576
K = 32
STEPS = ROWS // K
EROWS = 64


def _sc_gather_e(idx_e, E):
    mesh = plsc.VectorSubcoreMesh(core_axis_name="c", subcore_axis_name="s")

    @functools.partial(
        pl.kernel,
        out_type=jax.ShapeDtypeStruct((EROWS, EMB), jnp.float32),
        mesh=mesh,
        scratch_types=(
            pltpu.VMEM((8,), jnp.int32),
            pltpu.VMEM((8, EMB), jnp.float32),
            pltpu.SemaphoreType.DMA,
        ),
    )
    def k(idx_e_hbm, e_hbm, out_e, idx_e_v, rows_e, sem):
        wid = lax.axis_index("s") * 2 + lax.axis_index("c")

        @pl.when(wid < 8)
        def _():
            ebase = wid * 8
            pltpu.sync_copy(idx_e_hbm.at[pl.ds(ebase, 8)], idx_e_v)
            pltpu.async_copy(e_hbm.at[idx_e_v], rows_e, sem).wait()
            pltpu.sync_copy(rows_e, out_e.at[pl.ds(ebase, 8)])

    return k(idx_e, E)


def _feat_body(idx_sref, subs_ref, *refs):
    pm_blks = refs[0:K]
    ps_blks = refs[K:2 * K]
    r_out, q_out, tb_out = refs[2 * K:2 * K + 3]
    f32 = jnp.float32

    sub = subs_ref[...][:, 0:1]
    pm = jnp.concatenate([b[...][None] for b in pm_blks], axis=0)
    ps = jnp.concatenate([b[...][None] for b in ps_blks], axis=0)
    srow = lax.broadcasted_iota(jnp.int32, (K, 8), 1)
    onehot = (srow == sub).astype(f32)[:, :, None]
    m = jnp.sum(pm * onehot, axis=1)
    s = jnp.sum(ps * onehot, axis=1)

    v = s * s
    r = 1.0 / v
    q = m * r
    t = jnp.sum(m * q, axis=1, keepdims=True)
    b = 2.0 * jnp.sum(jnp.log(s), axis=1, keepdims=True)
    lane = lax.broadcasted_iota(jnp.int32, (K, CS), 1)
    tb = jnp.where(lane == 0, t, 0.0) + jnp.where(lane == 1, b, 0.0)
    r_out[...] = r
    q_out[...] = q
    tb_out[...] = tb


def _prior_features(idx_all, subs128, pm3, ps3):
    def mk_spec(k):
        return pl.BlockSpec(
            (8, CS), lambda i, idx, _k=k: (idx[K * i + _k] // 8, 0))

    grid_spec = pltpu.PrefetchScalarGridSpec(
        num_scalar_prefetch=1,
        grid=(STEPS,),
        in_specs=[pl.BlockSpec((K, 128), lambda i, idx: (i, 0))]
        + [mk_spec(k) for k in range(K)] * 2,
        out_specs=[pl.BlockSpec((K, CS), lambda i, idx: (i, 0))] * 3,
    )
    return pl.pallas_call(
        _feat_body,
        grid_spec=grid_spec,
        out_shape=[jax.ShapeDtypeStruct((ROWS, CS), jnp.float32)] * 3,
    )(idx_all, subs128, *([pm3] * K), *([ps3] * K))


def _combine_body(e_ref, r_ref, q_ref, tb_ref, mw_ref, mb_ref, uw_ref,
                  ub_ref, ww_ref, wb_ref, out_ref):
    f32 = jnp.float32
    hi = jax.lax.Precision.HIGHEST

    ex = e_ref[0:8, :]
    ec = e_ref[8:64, :]
    rw = jax.nn.relu(
        lax.dot_general(ex, mw_ref[...], (((1,), (1,)), ((), ())),
                        precision=hi, preferred_element_type=f32)
        + mb_ref[...])
    rc = jax.nn.relu(
        lax.dot_general(ec, mw_ref[...], (((1,), (1,)), ((), ())),
                        precision=hi, preferred_element_type=f32)
        + mb_ref[...])
    crow = lax.broadcasted_iota(jnp.int32, (56, CS), 0)
    rc = jnp.where(crow < CTX, rc, 0.0)
    h1 = CTX * rw[0:1, :]
    h2 = jnp.sum(rc, axis=0, keepdims=True)
    h = jnp.concatenate([h1, h2], axis=1)

    mu = lax.dot_general(h, uw_ref[...], (((1,), (1,)), ((), ())),
                         precision=hi, preferred_element_type=f32) + ub_ref[...]
    z = lax.dot_general(h, ww_ref[...], (((1,), (1,)), ((), ())),
                        precision=hi, preferred_element_type=f32) + wb_ref[...]
    post_var = jax.nn.softplus(z)
    log_post_var = jnp.sum(jnp.log(post_var))

    pvmu = post_var + mu * mu
    a1 = lax.dot_general(r_ref[...], pvmu, (((1,), (1,)), ((), ())),
                         precision=hi, preferred_element_type=f32)
    a2 = lax.dot_general(q_ref[...], mu, (((1,), (1,)), ((), ())),
                         precision=hi, preferred_element_type=f32)
    t = tb_ref[:, 0:1]
    b = tb_ref[:, 1:2]
    a = a1 - 2.0 * a2 + t
    kl = 0.5 * (a + b - CS - log_post_var)

    kl_x = kl[0:1, 0:1]
    kl_pos = kl[8:64, :]
    kl_neg = kl[64:576, :]

    irow = lax.broadcasted_iota(jnp.int32, (512, 56), 0)
    icol = lax.broadcasted_iota(jnp.int32, (512, 56), 1)
    sel = jnp.where(irow // NEG == icol, 1.0, 0.0).astype(f32)
    pos_for_neg = lax.dot_general(sel, kl_pos, (((1,), (0,)), ((), ())),
                                  precision=hi, preferred_element_type=f32)
    hinge = jnp.maximum(kl_neg - pos_for_neg + 1.0, 0.0)
    nrow = lax.broadcasted_iota(jnp.int32, (512, 1), 0)
    hinge = jnp.where(nrow < CTX * NEG, hinge, 0.0)
    likelihood = jnp.sum(hinge, keepdims=True)
    out_ref[...] = likelihood - kl_x


def kernel(x, context, neg_samples, E, M_w, M_b, U_w, U_b, W_w, W_b,
           prior_mus, prior_sigmas):
    zi = jnp.zeros((), jnp.int32)
    x = x.astype(jnp.int32)
    context = context.astype(jnp.int32)
    negf = neg_samples.reshape(-1).astype(jnp.int32)
    idx_all = jnp.concatenate([
        x, jnp.full((7,), zi), context, jnp.full((6,), zi),
        negf, jnp.full((12,), zi)])
    idx_e = idx_all[:EROWS]
    subs128 = jnp.broadcast_to((idx_all % 8)[:, None], (ROWS, 128))

    e_g = _sc_gather_e(idx_e, E)
    r_f, q_f, tb_f = _prior_features(idx_all, subs128, prior_mus,
                                     prior_sigmas)

    out = pl.pallas_call(
        _combine_body,
        out_shape=jax.ShapeDtypeStruct((1, 1), jnp.float32),
    )(e_g, r_f, q_f, tb_f,
      M_w, M_b.reshape(1, CS), U_w, U_b.reshape(1, CS),
      W_w, W_b.reshape(1, CS))
    return out.reshape((1,))


# speedup vs baseline: 1.3200x; 1.3200x over previous
import functools

import jax
import jax.numpy as jnp
from jax import lax
from jax.experimental import pallas as pl
from jax.experimental.pallas import tpu as pltpu
from jax.experimental.pallas import tpu_sc as plsc

VOCAB = 100000
EMB = 128
CS = 64
CTX = 50
NEG = 10

NW = 32
ROWS = 768
RPW = ROWS // NW
IDXPAD = 1024
EROWS = 64


def _sc_gather(idx_all, idx_e, pm, ps, E):
    mesh = plsc.VectorSubcoreMesh(core_axis_name="c", subcore_axis_name="s")

    @functools.partial(
        pl.kernel,
        out_type=(
            jax.ShapeDtypeStruct((ROWS, CS), jnp.float32),
            jax.ShapeDtypeStruct((ROWS, CS), jnp.float32),
            jax.ShapeDtypeStruct((EROWS, EMB), jnp.float32),
        ),
        mesh=mesh,
        scratch_types=(
            pltpu.VMEM((32,), jnp.int32),
            pltpu.VMEM((RPW, CS), jnp.float32),
            pltpu.VMEM((RPW, CS), jnp.float32),
            pltpu.VMEM((8,), jnp.int32),
            pltpu.VMEM((8, EMB), jnp.float32),
            pltpu.SemaphoreType.DMA,
            pltpu.SemaphoreType.DMA,
            pltpu.SemaphoreType.DMA,
        ),
    )
    def k(idx_hbm, idx_e_hbm, pm_hbm, ps_hbm, e_hbm, out_pm, out_ps, out_e,
          idx_v, rows_pm, rows_ps, idx_e_v, rows_e, sem0, sem1, sem2):
        wid = lax.axis_index("s") * 2 + lax.axis_index("c")
        base = wid * RPW
        pltpu.sync_copy(idx_hbm.at[pl.ds(base, 32)], idx_v)

        @pl.when(wid < 8)
        def _():
            ebase = wid * 8
            pltpu.sync_copy(idx_e_hbm.at[pl.ds(ebase, 8)], idx_e_v)
            pltpu.async_copy(e_hbm.at[idx_e_v], rows_e, sem2).wait()
            pltpu.sync_copy(rows_e, out_e.at[pl.ds(ebase, 8)])

        lo = idx_v[pl.ds(0, 16)]
        hi = idx_v[pl.ds(16, 16)]
        copies = []
        for j in range(RPW):
            vec = lo if j < 16 else hi
            r = vec[j % 16]
            copies.append(pltpu.async_copy(
                pm_hbm.at[pl.ds(r, 1)], rows_pm.at[pl.ds(j, 1)], sem0))
            copies.append(pltpu.async_copy(
                ps_hbm.at[pl.ds(r, 1)], rows_ps.at[pl.ds(j, 1)], sem1))
        for cp in copies:
            cp.wait()
        pltpu.sync_copy(rows_pm, out_pm.at[pl.ds(base, RPW)])
        pltpu.sync_copy(rows_ps, out_ps.at[pl.ds(base, RPW)])

    return k(idx_all, idx_e, pm, ps, E)


def _tc_body(pm_ref, ps_ref, e_ref, mw_ref, mb_ref, uw_ref, ub_ref,
             ww_ref, wb_ref, out_ref):
    f32 = jnp.float32
    hi = jax.lax.Precision.HIGHEST

    ex = e_ref[0:8, :]
    ec = e_ref[8:64, :]
    rw = jax.nn.relu(
        lax.dot_general(ex, mw_ref[...], (((1,), (1,)), ((), ())),
                        precision=hi, preferred_element_type=f32)
        + mb_ref[...])
    rc = jax.nn.relu(
        lax.dot_general(ec, mw_ref[...], (((1,), (1,)), ((), ())),
                        precision=hi, preferred_element_type=f32)
        + mb_ref[...])
    crow = lax.broadcasted_iota(jnp.int32, (56, CS), 0)
    rc = jnp.where(crow < CTX, rc, 0.0)
    h1 = CTX * rw[0:1, :]
    h2 = jnp.sum(rc, axis=0, keepdims=True)
    h = jnp.concatenate([h1, h2], axis=1)

    mu = lax.dot_general(h, uw_ref[...], (((1,), (1,)), ((), ())),
                         precision=hi, preferred_element_type=f32) + ub_ref[...]
    z = lax.dot_general(h, ww_ref[...], (((1,), (1,)), ((), ())),
                        precision=hi, preferred_element_type=f32) + wb_ref[...]
    post_var = jax.nn.softplus(z)
    log_post_var = jnp.sum(jnp.log(post_var))

    pm = pm_ref[...]
    ps = ps_ref[...]
    v = ps * ps
    a = jnp.sum((post_var + (pm - mu) ** 2) / v, axis=1, keepdims=True)
    b = 2.0 * jnp.sum(jnp.log(ps), axis=1, keepdims=True)
    kl = 0.5 * (a + b - CS - log_post_var)

    kl_x = kl[0:1, 0:1]
    kl_pos = kl[64:128, :]
    kl_neg = kl[256:768, :]

    irow = lax.broadcasted_iota(jnp.int32, (512, 64), 0)
    icol = lax.broadcasted_iota(jnp.int32, (512, 64), 1)
    sel = jnp.where(irow // NEG == icol, 1.0, 0.0).astype(f32)
    pos_for_neg = lax.dot_general(sel, kl_pos, (((1,), (0,)), ((), ())),
                                  precision=hi, preferred_element_type=f32)
    hinge = jnp.maximum(kl_neg - pos_for_neg + 1.0, 0.0)
    nrow = lax.broadcasted_iota(jnp.int32, (512, 1), 0)
    hinge = jnp.where(nrow < CTX * NEG, hinge, 0.0)
    likelihood = jnp.sum(hinge, keepdims=True)
    out_ref[...] = likelihood - kl_x


def kernel(x, context, neg_samples, E, M_w, M_b, U_w, U_b, W_w, W_b,
           prior_mus, prior_sigmas):
    zi = jnp.zeros((), jnp.int32)
    x = x.astype(jnp.int32)
    context = context.astype(jnp.int32)
    negf = neg_samples.reshape(-1).astype(jnp.int32)
    idx_all = jnp.concatenate([
        x, jnp.full((63,), zi), context, jnp.full((142,), zi),
        negf, jnp.full((IDXPAD - 756,), zi)])
    idx_e = jnp.concatenate([x, jnp.full((7,), zi), context,
                             jnp.full((6,), zi)])

    pm_g, ps_g, e_g = _sc_gather(idx_all, idx_e, prior_mus, prior_sigmas, E)

    out = pl.pallas_call(
        _tc_body,
        out_shape=jax.ShapeDtypeStruct((1, 1), jnp.float32),
    )(pm_g, ps_g, e_g,
      M_w, M_b.reshape(1, CS), U_w, U_b.reshape(1, CS),
      W_w, W_b.reshape(1, CS))
    return out.reshape((1,))


# speedup vs baseline: 1.3405x; 1.0155x over previous
import functools

import jax
import jax.numpy as jnp
from jax import lax
from jax.experimental import pallas as pl
from jax.experimental.pallas import tpu as pltpu
from jax.experimental.pallas import tpu_sc as plsc

VOCAB = 100000
EMB = 128
CS = 64
CTX = 50
NEG = 10

NW = 32
ROWS = 768
RPW = ROWS // NW
IDXPAD = 1024
EROWS = 64


def _sc_gather(idx_all, idx_e, pm, ps, E):
    mesh = plsc.VectorSubcoreMesh(core_axis_name="c", subcore_axis_name="s")

    @functools.partial(
        pl.kernel,
        out_type=(
            jax.ShapeDtypeStruct((ROWS, CS), jnp.float32),
            jax.ShapeDtypeStruct((ROWS, CS), jnp.float32),
            jax.ShapeDtypeStruct((EROWS, EMB), jnp.float32),
        ),
        mesh=mesh,
        compiler_params=pltpu.CompilerParams(use_tc_tiling_on_sc=True),
        scratch_types=(
            pltpu.VMEM((32,), jnp.int32),
            pltpu.VMEM((RPW, CS), jnp.float32),
            pltpu.VMEM((RPW, CS), jnp.float32),
            pltpu.VMEM((8,), jnp.int32),
            pltpu.VMEM((8, EMB), jnp.float32),
            pltpu.SemaphoreType.DMA,
            pltpu.SemaphoreType.DMA,
            pltpu.SemaphoreType.DMA,
        ),
    )
    def k(idx_hbm, idx_e_hbm, pm_hbm, ps_hbm, e_hbm, out_pm, out_ps, out_e,
          idx_v, rows_pm, rows_ps, idx_e_v, rows_e, sem0, sem1, sem2):
        wid = lax.axis_index("s") * 2 + lax.axis_index("c")
        base = wid * RPW
        pltpu.sync_copy(idx_hbm.at[pl.ds(base, 32)], idx_v)

        @pl.when(wid < 8)
        def _():
            ebase = wid * 8
            pltpu.sync_copy(idx_e_hbm.at[pl.ds(ebase, 8)], idx_e_v)
            pltpu.async_copy(e_hbm.at[idx_e_v], rows_e, sem2).wait()
            pltpu.sync_copy(rows_e, out_e.at[pl.ds(ebase, 8)])

        lo = idx_v[pl.ds(0, 16)]
        hi = idx_v[pl.ds(16, 16)]
        copies = []
        for j in range(RPW):
            vec = lo if j < 16 else hi
            r = vec[j % 16]
            copies.append(pltpu.async_copy(
                pm_hbm.at[pl.ds(r, 1)], rows_pm.at[pl.ds(j, 1)], sem0))
            copies.append(pltpu.async_copy(
                ps_hbm.at[pl.ds(r, 1)], rows_ps.at[pl.ds(j, 1)], sem1))
        for cp in copies:
            cp.wait()
        pltpu.sync_copy(rows_pm, out_pm.at[pl.ds(base, RPW)])
        pltpu.sync_copy(rows_ps, out_ps.at[pl.ds(base, RPW)])

    return k(idx_all, idx_e, pm, ps, E)


def _tc_body(pm_ref, ps_ref, e_ref, mw_ref, mb_ref, uw_ref, ub_ref,
             ww_ref, wb_ref, out_ref):
    f32 = jnp.float32
    hi = jax.lax.Precision.HIGHEST

    ex = e_ref[0:8, :]
    ec = e_ref[8:64, :]
    rw = jax.nn.relu(
        lax.dot_general(ex, mw_ref[...], (((1,), (1,)), ((), ())),
                        precision=hi, preferred_element_type=f32)
        + mb_ref[...])
    rc = jax.nn.relu(
        lax.dot_general(ec, mw_ref[...], (((1,), (1,)), ((), ())),
                        precision=hi, preferred_element_type=f32)
        + mb_ref[...])
    crow = lax.broadcasted_iota(jnp.int32, (56, CS), 0)
    rc = jnp.where(crow < CTX, rc, 0.0)
    h1 = CTX * rw[0:1, :]
    h2 = jnp.sum(rc, axis=0, keepdims=True)
    h = jnp.concatenate([h1, h2], axis=1)

    mu = lax.dot_general(h, uw_ref[...], (((1,), (1,)), ((), ())),
                         precision=hi, preferred_element_type=f32) + ub_ref[...]
    z = lax.dot_general(h, ww_ref[...], (((1,), (1,)), ((), ())),
                        precision=hi, preferred_element_type=f32) + wb_ref[...]
    post_var = jax.nn.softplus(z)
    log_post_var = jnp.sum(jnp.log(post_var))

    pm = pm_ref[...]
    ps = ps_ref[...]
    v = ps * ps
    a = jnp.sum((post_var + (pm - mu) ** 2) / v, axis=1, keepdims=True)
    b = 2.0 * jnp.sum(jnp.log(ps), axis=1, keepdims=True)
    kl = 0.5 * (a + b - CS - log_post_var)

    kl_x = kl[0:1, 0:1]
    kl_pos = kl[64:128, :]
    kl_neg = kl[256:768, :]

    irow = lax.broadcasted_iota(jnp.int32, (512, 64), 0)
    icol = lax.broadcasted_iota(jnp.int32, (512, 64), 1)
    sel = jnp.where(irow // NEG == icol, 1.0, 0.0).astype(f32)
    pos_for_neg = lax.dot_general(sel, kl_pos, (((1,), (0,)), ((), ())),
                                  precision=hi, preferred_element_type=f32)
    hinge = jnp.maximum(kl_neg - pos_for_neg + 1.0, 0.0)
    nrow = lax.broadcasted_iota(jnp.int32, (512, 1), 0)
    hinge = jnp.where(nrow < CTX * NEG, hinge, 0.0)
    likelihood = jnp.sum(hinge, keepdims=True)
    out_ref[...] = likelihood - kl_x


def kernel(x, context, neg_samples, E, M_w, M_b, U_w, U_b, W_w, W_b,
           prior_mus, prior_sigmas):
    zi = jnp.zeros((), jnp.int32)
    x = x.astype(jnp.int32)
    context = context.astype(jnp.int32)
    negf = neg_samples.reshape(-1).astype(jnp.int32)
    idx_all = jnp.concatenate([
        x, jnp.full((63,), zi), context, jnp.full((142,), zi),
        negf, jnp.full((IDXPAD - 756,), zi)])
    idx_e = jnp.concatenate([x, jnp.full((7,), zi), context,
                             jnp.full((6,), zi)])

    pm_g, ps_g, e_g = _sc_gather(idx_all, idx_e, prior_mus, prior_sigmas, E)

    out = pl.pallas_call(
        _tc_body,
        out_shape=jax.ShapeDtypeStruct((1, 1), jnp.float32),
    )(pm_g, ps_g, e_g,
      M_w, M_b.reshape(1, CS), U_w, U_b.reshape(1, CS),
      W_w, W_b.reshape(1, CS))
    return out.reshape((1,))


# speedup vs baseline: 1.3964x; 1.0417x over previous
import functools

import jax
import jax.numpy as jnp
from jax import lax
from jax.experimental import pallas as pl
from jax.experimental.pallas import tpu as pltpu
from jax.experimental.pallas import tpu_sc as plsc

VOCAB = 100000
EMB = 128
CS = 64
CTX = 50
NEG = 10

ROWS = 768
EROWS = 64
SEGS = ((0, 8), (64, 56), (256, 504))


def _sc_gather_e(idx_e, E):
    mesh = plsc.VectorSubcoreMesh(core_axis_name="c", subcore_axis_name="s")

    @functools.partial(
        pl.kernel,
        out_type=jax.ShapeDtypeStruct((EROWS, EMB), jnp.float32),
        mesh=mesh,
        scratch_types=(
            pltpu.VMEM((8,), jnp.int32),
            pltpu.VMEM((8, EMB), jnp.float32),
            pltpu.SemaphoreType.DMA,
        ),
    )
    def k(idx_e_hbm, e_hbm, out_e, idx_e_v, rows_e, sem):
        wid = lax.axis_index("s") * 2 + lax.axis_index("c")

        @pl.when(wid < 8)
        def _():
            ebase = wid * 8
            pltpu.sync_copy(idx_e_hbm.at[pl.ds(ebase, 8)], idx_e_v)
            pltpu.async_copy(e_hbm.at[idx_e_v], rows_e, sem).wait()
            pltpu.sync_copy(rows_e, out_e.at[pl.ds(ebase, 8)])

    return k(idx_e, E)


def _gather_body(idx_ref, pm_any, ps_any, pm_out, ps_out, sem0, sem1):
    pm_out[...] = jnp.ones((ROWS, CS), jnp.float32)
    ps_out[...] = jnp.ones((ROWS, CS), jnp.float32)

    def fire(j, carry):
        r = idx_ref[j]
        pltpu.make_async_copy(
            pm_any.at[pl.ds(r, 1)], pm_out.at[pl.ds(j, 1)], sem0).start()
        pltpu.make_async_copy(
            ps_any.at[pl.ds(r, 1)], ps_out.at[pl.ds(j, 1)], sem1).start()
        return carry

    for start, size in SEGS:
        lax.fori_loop(start, start + size, fire, 0)
    for start, size in SEGS:
        pltpu.make_async_copy(
            pm_any.at[pl.ds(0, size)], pm_out.at[pl.ds(start, size)],
            sem0).wait()
        pltpu.make_async_copy(
            ps_any.at[pl.ds(0, size)], ps_out.at[pl.ds(start, size)],
            sem1).wait()


def _prior_gather(idx_all, pm, ps):
    return pl.pallas_call(
        _gather_body,
        in_specs=[
            pl.BlockSpec(memory_space=pltpu.SMEM),
            pl.BlockSpec(memory_space=pl.ANY),
            pl.BlockSpec(memory_space=pl.ANY),
        ],
        out_shape=[jax.ShapeDtypeStruct((ROWS, CS), jnp.float32)] * 2,
        scratch_shapes=[pltpu.SemaphoreType.DMA, pltpu.SemaphoreType.DMA],
    )(idx_all, pm, ps)


def _tc_body(pm_ref, ps_ref, e_ref, mw_ref, mb_ref, uw_ref, ub_ref,
             ww_ref, wb_ref, out_ref):
    f32 = jnp.float32
    hi = jax.lax.Precision.HIGHEST

    ex = e_ref[0:8, :]
    ec = e_ref[8:64, :]
    rw = jax.nn.relu(
        lax.dot_general(ex, mw_ref[...], (((1,), (1,)), ((), ())),
                        precision=hi, preferred_element_type=f32)
        + mb_ref[...])
    rc = jax.nn.relu(
        lax.dot_general(ec, mw_ref[...], (((1,), (1,)), ((), ())),
                        precision=hi, preferred_element_type=f32)
        + mb_ref[...])
    crow = lax.broadcasted_iota(jnp.int32, (56, CS), 0)
    rc = jnp.where(crow < CTX, rc, 0.0)
    h1 = CTX * rw[0:1, :]
    h2 = jnp.sum(rc, axis=0, keepdims=True)
    h = jnp.concatenate([h1, h2], axis=1)

    mu = lax.dot_general(h, uw_ref[...], (((1,), (1,)), ((), ())),
                         precision=hi, preferred_element_type=f32) + ub_ref[...]
    z = lax.dot_general(h, ww_ref[...], (((1,), (1,)), ((), ())),
                        precision=hi, preferred_element_type=f32) + wb_ref[...]
    post_var = jax.nn.softplus(z)
    log_post_var = jnp.sum(jnp.log(post_var))

    pm = pm_ref[...]
    ps = ps_ref[...]
    v = ps * ps
    a = jnp.sum((post_var + (pm - mu) ** 2) / v, axis=1, keepdims=True)
    b = 2.0 * jnp.sum(jnp.log(ps), axis=1, keepdims=True)
    kl = 0.5 * (a + b - CS - log_post_var)

    kl_x = kl[0:1, 0:1]
    kl_pos = kl[64:128, :]
    kl_neg = kl[256:768, :]

    irow = lax.broadcasted_iota(jnp.int32, (512, 64), 0)
    icol = lax.broadcasted_iota(jnp.int32, (512, 64), 1)
    sel = jnp.where(irow // NEG == icol, 1.0, 0.0).astype(f32)
    pos_for_neg = lax.dot_general(sel, kl_pos, (((1,), (0,)), ((), ())),
                                  precision=hi, preferred_element_type=f32)
    hinge = jnp.maximum(kl_neg - pos_for_neg + 1.0, 0.0)
    nrow = lax.broadcasted_iota(jnp.int32, (512, 1), 0)
    hinge = jnp.where(nrow < CTX * NEG, hinge, 0.0)
    likelihood = jnp.sum(hinge, keepdims=True)
    out_ref[...] = likelihood - kl_x


def kernel(x, context, neg_samples, E, M_w, M_b, U_w, U_b, W_w, W_b,
           prior_mus, prior_sigmas):
    zi = jnp.zeros((), jnp.int32)
    x = x.astype(jnp.int32)
    context = context.astype(jnp.int32)
    negf = neg_samples.reshape(-1).astype(jnp.int32)
    idx_all = jnp.concatenate([
        x, jnp.full((63,), zi), context, jnp.full((142,), zi),
        negf, jnp.full((12,), zi)])
    idx_e = jnp.concatenate([x, jnp.full((7,), zi), context,
                             jnp.full((6,), zi)])

    e_g = _sc_gather_e(idx_e, E)
    pm_g, ps_g = _prior_gather(idx_all, prior_mus, prior_sigmas)

    out = pl.pallas_call(
        _tc_body,
        out_shape=jax.ShapeDtypeStruct((1, 1), jnp.float32),
    )(pm_g, ps_g, e_g,
      M_w, M_b.reshape(1, CS), U_w, U_b.reshape(1, CS),
      W_w, W_b.reshape(1, CS))
    return out.reshape((1,))


# speedup vs baseline: 1.5248x; 1.0920x over previous
import functools

import jax
import jax.numpy as jnp
from jax import lax
from jax.experimental import pallas as pl
from jax.experimental.pallas import tpu as pltpu
from jax.experimental.pallas import tpu_sc as plsc

VOCAB = 100000
EMB = 128
CS = 64
CTX = 50
NEG = 10

ROWS = 1024
RPW = ROWS // 32
EROWS = 64
BLK = 2048
NBLK = 50
FROWS = NBLK * BLK // 128


def _sweep_body(idx_e_ref, e_any, pmt_ref, pst_ref, mw_ref, mbc_ref,
                uw_ref, ubc_ref, ww_ref, wbc_ref, a_ref, b_ref, mupv_ref,
                ev, sem):
    f32 = jnp.float32
    hi = jax.lax.Precision.HIGHEST
    i = pl.program_id(0)

    @pl.when(i == 0)
    def _mlp():
        def fire(j, carry):
            r = idx_e_ref[j]
            pltpu.make_async_copy(
                e_any.at[pl.ds(r, 1)], ev.at[pl.ds(j, 1)], sem).start()
            return carry
        lax.fori_loop(0, EROWS, fire, 0)
        pltpu.make_async_copy(e_any.at[pl.ds(0, EROWS)], ev, sem).wait()

        e = ev[...]
        r128 = lax.broadcasted_iota(jnp.int32, (EMB, EMB), 0)
        c128 = lax.broadcasted_iota(jnp.int32, (EMB, EMB), 1)
        ident = jnp.where(r128 == c128, 1.0, 0.0).astype(f32)
        ext = lax.dot_general(ident, e[0:8, :], (((1,), (1,)), ((), ())),
                              precision=hi, preferred_element_type=f32)
        ect = lax.dot_general(ident, e[8:64, :], (((1,), (1,)), ((), ())),
                              precision=hi, preferred_element_type=f32)
        ex_col = ext[:, 0:1]
        rw = jax.nn.relu(
            lax.dot_general(mw_ref[...], ex_col, (((1,), (0,)), ((), ())),
                            precision=hi, preferred_element_type=f32)
            + mbc_ref[...])
        rcm = jax.nn.relu(
            lax.dot_general(mw_ref[...], ect, (((1,), (0,)), ((), ())),
                            precision=hi, preferred_element_type=f32)
            + mbc_ref[...])
        ccol = lax.broadcasted_iota(jnp.int32, (CS, 56), 1)
        rcm = jnp.where(ccol < CTX, rcm, 0.0)
        h2 = jnp.sum(rcm, axis=1, keepdims=True)
        h = jnp.concatenate([CTX * rw, h2], axis=0)
        mu = lax.dot_general(uw_ref[...], h, (((1,), (0,)), ((), ())),
                             precision=hi, preferred_element_type=f32) \
            + ubc_ref[...]
        z = lax.dot_general(ww_ref[...], h, (((1,), (0,)), ((), ())),
                            precision=hi, preferred_element_type=f32) \
            + wbc_ref[...]
        pv = jax.nn.softplus(z)
        lpv = jnp.sum(jnp.log(pv))
        lane8 = lax.broadcasted_iota(jnp.int32, (CS, 8), 1)
        mupv_ref[...] = (
            jnp.where(lane8 == 0, jnp.broadcast_to(mu, (CS, 8)), 0.0)
            + jnp.where(lane8 == 1, jnp.broadcast_to(pv, (CS, 8)), 0.0)
            + jnp.where(lane8 == 2, lpv, 0.0))

    mu_col = mupv_ref[:, 0:1]
    pv_col = mupv_ref[:, 1:2]
    pm = pmt_ref[...]
    s = pst_ref[...]
    v = s * s
    d = pm - mu_col
    a_part = jnp.sum((pv_col + d * d) / v, axis=0, keepdims=True)
    p = s
    for half in (32, 16, 8, 4, 2, 1):
        p = p[0:half, :] * p[half:2 * half, :]
    b_part = 2.0 * jnp.log(p)
    a_ref[...] = jnp.concatenate(
        [a_part[:, k * 128:(k + 1) * 128] for k in range(BLK // 128)], axis=0)
    b_ref[...] = jnp.concatenate(
        [b_part[:, k * 128:(k + 1) * 128] for k in range(BLK // 128)], axis=0)


def _sweep(idx_e, E, pmT, psT, M_w, M_bc, U_w, U_bc, W_w, W_bc):
    last = VOCAB // BLK
    return pl.pallas_call(
        _sweep_body,
        grid=(NBLK,),
        in_specs=[
            pl.BlockSpec(memory_space=pltpu.SMEM),
            pl.BlockSpec(memory_space=pl.ANY),
            pl.BlockSpec((CS, BLK), lambda i: (0, jnp.minimum(i, last))),
            pl.BlockSpec((CS, BLK), lambda i: (0, jnp.minimum(i, last))),
            pl.BlockSpec((CS, EMB), lambda i: (0, 0)),
            pl.BlockSpec((CS, 1), lambda i: (0, 0)),
            pl.BlockSpec((CS, EMB), lambda i: (0, 0)),
            pl.BlockSpec((CS, 1), lambda i: (0, 0)),
            pl.BlockSpec((CS, EMB), lambda i: (0, 0)),
            pl.BlockSpec((CS, 1), lambda i: (0, 0)),
        ],
        out_specs=[
            pl.BlockSpec((BLK // 128, 128), lambda i: (i, 0)),
            pl.BlockSpec((BLK // 128, 128), lambda i: (i, 0)),
            pl.BlockSpec((CS, 8), lambda i: (0, 0)),
        ],
        out_shape=[
            jax.ShapeDtypeStruct((FROWS, 128), jnp.float32),
            jax.ShapeDtypeStruct((FROWS, 128), jnp.float32),
            jax.ShapeDtypeStruct((CS, 8), jnp.float32),
        ],
        scratch_shapes=[pltpu.VMEM((EROWS, EMB), jnp.float32),
                        pltpu.SemaphoreType.DMA],
    )(idx_e, E, pmT, psT, M_w, M_bc, U_w, U_bc, W_w, W_bc)


def _sc_gather(idx_all, a_arr, b_arr):
    mesh = plsc.VectorSubcoreMesh(core_axis_name="c", subcore_axis_name="s")

    @functools.partial(
        pl.kernel,
        out_type=(
            jax.ShapeDtypeStruct((ROWS, 128), jnp.float32),
            jax.ShapeDtypeStruct((ROWS, 128), jnp.float32),
        ),
        mesh=mesh,
        scratch_types=(
            pltpu.VMEM((RPW,), jnp.int32),
            pltpu.VMEM((RPW,), jnp.int32),
            pltpu.VMEM((RPW, 128), jnp.float32),
            pltpu.VMEM((RPW, 128), jnp.float32),
            pltpu.SemaphoreType.DMA,
            pltpu.SemaphoreType.DMA,
        ),
    )
    def k(idx_hbm, a_hbm, b_hbm, out_a, out_b, idxv, tilev, rows_a, rows_b,
          sem0, sem1):
        wid = lax.axis_index("s") * 2 + lax.axis_index("c")
        base = wid * RPW
        pltpu.sync_copy(idx_hbm.at[pl.ds(base, RPW)], idxv)
        for g in range(RPW // 16):
            iv = idxv[pl.ds(16 * g, 16)]
            tilev[pl.ds(16 * g, 16)] = lax.shift_right_logical(iv, 7)
        cp0 = pltpu.async_copy(a_hbm.at[tilev], rows_a, sem0)
        cp1 = pltpu.async_copy(b_hbm.at[tilev], rows_b, sem1)
        cp0.wait()
        pltpu.sync_copy(rows_a, out_a.at[pl.ds(base, RPW)])
        cp1.wait()
        pltpu.sync_copy(rows_b, out_b.at[pl.ds(base, RPW)])

    return k(idx_all, a_arr, b_arr)


def _hinge_body(ar_ref, br_ref, sub_ref, mupv_ref, out_ref):
    f32 = jnp.float32
    hi = jax.lax.Precision.HIGHEST
    lpv = mupv_ref[0:1, 2:3]
    lane = lax.broadcasted_iota(jnp.int32, (ROWS, 128), 1)
    onehot = jnp.where(lane == sub_ref[...], 1.0, 0.0).astype(f32)
    a = jnp.sum(ar_ref[...] * onehot, axis=1, keepdims=True)
    b = jnp.sum(br_ref[...] * onehot, axis=1, keepdims=True)
    kl = 0.5 * (a + b - CS - lpv)

    kl_x = kl[0:1, 0:1]
    kl_pos = kl[64:128, :]
    kl_neg = kl[256:768, :]
    irow = lax.broadcasted_iota(jnp.int32, (512, 64), 0)
    icol = lax.broadcasted_iota(jnp.int32, (512, 64), 1)
    sel = jnp.where(irow // NEG == icol, 1.0, 0.0).astype(f32)
    pos_for_neg = lax.dot_general(sel, kl_pos, (((1,), (0,)), ((), ())),
                                  precision=hi, preferred_element_type=f32)
    hinge = jnp.maximum(kl_neg - pos_for_neg + 1.0, 0.0)
    nrow = lax.broadcasted_iota(jnp.int32, (512, 1), 0)
    hinge = jnp.where(nrow < CTX * NEG, hinge, 0.0)
    out_ref[...] = jnp.sum(hinge, keepdims=True) - kl_x


def kernel(x, context, neg_samples, E, M_w, M_b, U_w, U_b, W_w, W_b,
           prior_mus, prior_sigmas):
    zi = jnp.zeros((), jnp.int32)
    x = x.astype(jnp.int32)
    context = context.astype(jnp.int32)
    negf = neg_samples.reshape(-1).astype(jnp.int32)
    idx_all = jnp.concatenate([
        x, jnp.full((63,), zi), context, jnp.full((142,), zi),
        negf, jnp.full((ROWS - 756,), zi)])
    idx_e = jnp.concatenate([x, jnp.full((7,), zi), context,
                             jnp.full((6,), zi)])
    sub = (idx_all & 127).reshape(ROWS, 1)

    a_arr, b_arr, mupv = _sweep(
        idx_e, E, prior_mus.T, prior_sigmas.T,
        M_w, M_b.reshape(CS, 1), U_w, U_b.reshape(CS, 1),
        W_w, W_b.reshape(CS, 1))
    ar, br = _sc_gather(idx_all, a_arr, b_arr)
    out = pl.pallas_call(
        _hinge_body,
        out_shape=jax.ShapeDtypeStruct((1, 1), jnp.float32),
    )(ar, br, sub, mupv)
    return out.reshape((1,))


# speedup vs baseline: 1.7452x; 1.1445x over previous
import functools

import jax
import jax.numpy as jnp
from jax import lax
from jax.experimental import pallas as pl
from jax.experimental.pallas import tpu as pltpu
from jax.experimental.pallas import tpu_sc as plsc

VOCAB = 100000
EMB = 128
CS = 64
CTX = 50
NEG = 10

ROWS = 1024
RPW = ROWS // 32
EROWS = 64
BLK = 4096
NBLK = 25
FROWS = NBLK * BLK // 128


def _sweep_body(idx_e_ref, e_any, pmt_ref, pst_ref, mw_ref, mbc_ref,
                uw_ref, ubc_ref, ww_ref, wbc_ref, a_ref, b_ref, mupv_ref,
                ev, sem):
    f32 = jnp.float32
    hi = jax.lax.Precision.HIGHEST
    i = pl.program_id(0)

    @pl.when(i == 0)
    def _mlp():
        def fire(j, carry):
            r = idx_e_ref[j]
            pltpu.make_async_copy(
                e_any.at[pl.ds(r, 1)], ev.at[pl.ds(j, 1)], sem).start()
            return carry
        lax.fori_loop(0, EROWS, fire, 0)
        pltpu.make_async_copy(e_any.at[pl.ds(0, EROWS)], ev, sem).wait()

        e = ev[...]
        r128 = lax.broadcasted_iota(jnp.int32, (EMB, EMB), 0)
        c128 = lax.broadcasted_iota(jnp.int32, (EMB, EMB), 1)
        ident = jnp.where(r128 == c128, 1.0, 0.0).astype(f32)
        ext = lax.dot_general(ident, e[0:8, :], (((1,), (1,)), ((), ())),
                              precision=hi, preferred_element_type=f32)
        ect = lax.dot_general(ident, e[8:64, :], (((1,), (1,)), ((), ())),
                              precision=hi, preferred_element_type=f32)
        ex_col = ext[:, 0:1]
        rw = jax.nn.relu(
            lax.dot_general(mw_ref[...], ex_col, (((1,), (0,)), ((), ())),
                            precision=hi, preferred_element_type=f32)
            + mbc_ref[...])
        rcm = jax.nn.relu(
            lax.dot_general(mw_ref[...], ect, (((1,), (0,)), ((), ())),
                            precision=hi, preferred_element_type=f32)
            + mbc_ref[...])
        ccol = lax.broadcasted_iota(jnp.int32, (CS, 56), 1)
        rcm = jnp.where(ccol < CTX, rcm, 0.0)
        h2 = jnp.sum(rcm, axis=1, keepdims=True)
        h = jnp.concatenate([CTX * rw, h2], axis=0)
        mu = lax.dot_general(uw_ref[...], h, (((1,), (0,)), ((), ())),
                             precision=hi, preferred_element_type=f32) \
            + ubc_ref[...]
        z = lax.dot_general(ww_ref[...], h, (((1,), (0,)), ((), ())),
                            precision=hi, preferred_element_type=f32) \
            + wbc_ref[...]
        pv = jax.nn.softplus(z)
        lpv = jnp.sum(jnp.log(pv))
        lane8 = lax.broadcasted_iota(jnp.int32, (CS, 8), 1)
        mupv_ref[...] = (
            jnp.where(lane8 == 0, jnp.broadcast_to(mu, (CS, 8)), 0.0)
            + jnp.where(lane8 == 1, jnp.broadcast_to(pv, (CS, 8)), 0.0)
            + jnp.where(lane8 == 2, lpv, 0.0))

    mu_col = mupv_ref[:, 0:1]
    pv_col = mupv_ref[:, 1:2]
    pm = pmt_ref[...]
    s = pst_ref[...]
    v = s * s
    d = pm - mu_col
    a_part = jnp.sum((pv_col + d * d) / v, axis=0, keepdims=True)
    p = s
    for half in (32, 16, 8, 4, 2, 1):
        p = p[0:half, :] * p[half:2 * half, :]
    b_part = 2.0 * jnp.log(p)
    a_ref[...] = jnp.concatenate(
        [a_part[:, k * 128:(k + 1) * 128] for k in range(BLK // 128)], axis=0)
    b_ref[...] = jnp.concatenate(
        [b_part[:, k * 128:(k + 1) * 128] for k in range(BLK // 128)], axis=0)


def _sweep(idx_e, E, pmT, psT, M_w, M_bc, U_w, U_bc, W_w, W_bc):
    last = VOCAB // BLK
    return pl.pallas_call(
        _sweep_body,
        grid=(NBLK,),
        in_specs=[
            pl.BlockSpec(memory_space=pltpu.SMEM),
            pl.BlockSpec(memory_space=pl.ANY),
            pl.BlockSpec((CS, BLK), lambda i: (0, jnp.minimum(i, last))),
            pl.BlockSpec((CS, BLK), lambda i: (0, jnp.minimum(i, last))),
            pl.BlockSpec((CS, EMB), lambda i: (0, 0)),
            pl.BlockSpec((CS, 1), lambda i: (0, 0)),
            pl.BlockSpec((CS, EMB), lambda i: (0, 0)),
            pl.BlockSpec((CS, 1), lambda i: (0, 0)),
            pl.BlockSpec((CS, EMB), lambda i: (0, 0)),
            pl.BlockSpec((CS, 1), lambda i: (0, 0)),
        ],
        out_specs=[
            pl.BlockSpec((BLK // 128, 128), lambda i: (i, 0)),
            pl.BlockSpec((BLK // 128, 128), lambda i: (i, 0)),
            pl.BlockSpec((CS, 8), lambda i: (0, 0)),
        ],
        out_shape=[
            jax.ShapeDtypeStruct((FROWS, 128), jnp.float32),
            jax.ShapeDtypeStruct((FROWS, 128), jnp.float32),
            jax.ShapeDtypeStruct((CS, 8), jnp.float32),
        ],
        scratch_shapes=[pltpu.VMEM((EROWS, EMB), jnp.float32),
                        pltpu.SemaphoreType.DMA],
    )(idx_e, E, pmT, psT, M_w, M_bc, U_w, U_bc, W_w, W_bc)


def _sc_gather(idx_all, a_arr, b_arr):
    mesh = plsc.VectorSubcoreMesh(core_axis_name="c", subcore_axis_name="s")

    @functools.partial(
        pl.kernel,
        out_type=(
            jax.ShapeDtypeStruct((ROWS, 128), jnp.float32),
            jax.ShapeDtypeStruct((ROWS, 128), jnp.float32),
        ),
        mesh=mesh,
        scratch_types=(
            pltpu.VMEM((RPW,), jnp.int32),
            pltpu.VMEM((RPW,), jnp.int32),
            pltpu.VMEM((RPW, 128), jnp.float32),
            pltpu.VMEM((RPW, 128), jnp.float32),
            pltpu.SemaphoreType.DMA,
            pltpu.SemaphoreType.DMA,
        ),
    )
    def k(idx_hbm, a_hbm, b_hbm, out_a, out_b, idxv, tilev, rows_a, rows_b,
          sem0, sem1):
        wid = lax.axis_index("s") * 2 + lax.axis_index("c")
        base = wid * RPW
        pltpu.sync_copy(idx_hbm.at[pl.ds(base, RPW)], idxv)
        for g in range(RPW // 16):
            iv = idxv[pl.ds(16 * g, 16)]
            tilev[pl.ds(16 * g, 16)] = lax.shift_right_logical(iv, 7)
        cp0 = pltpu.async_copy(a_hbm.at[tilev], rows_a, sem0)
        cp1 = pltpu.async_copy(b_hbm.at[tilev], rows_b, sem1)
        cp0.wait()
        pltpu.sync_copy(rows_a, out_a.at[pl.ds(base, RPW)])
        cp1.wait()
        pltpu.sync_copy(rows_b, out_b.at[pl.ds(base, RPW)])

    return k(idx_all, a_arr, b_arr)


def _hinge_body(ar_ref, br_ref, sub_ref, mupv_ref, out_ref):
    f32 = jnp.float32
    hi = jax.lax.Precision.HIGHEST
    lpv = mupv_ref[0:1, 2:3]
    lane = lax.broadcasted_iota(jnp.int32, (ROWS, 128), 1)
    onehot = jnp.where(lane == sub_ref[...], 1.0, 0.0).astype(f32)
    a = jnp.sum(ar_ref[...] * onehot, axis=1, keepdims=True)
    b = jnp.sum(br_ref[...] * onehot, axis=1, keepdims=True)
    kl = 0.5 * (a + b - CS - lpv)

    kl_x = kl[0:1, 0:1]
    kl_pos = kl[64:128, :]
    kl_neg = kl[256:768, :]
    irow = lax.broadcasted_iota(jnp.int32, (512, 64), 0)
    icol = lax.broadcasted_iota(jnp.int32, (512, 64), 1)
    sel = jnp.where(irow // NEG == icol, 1.0, 0.0).astype(f32)
    pos_for_neg = lax.dot_general(sel, kl_pos, (((1,), (0,)), ((), ())),
                                  precision=hi, preferred_element_type=f32)
    hinge = jnp.maximum(kl_neg - pos_for_neg + 1.0, 0.0)
    nrow = lax.broadcasted_iota(jnp.int32, (512, 1), 0)
    hinge = jnp.where(nrow < CTX * NEG, hinge, 0.0)
    out_ref[...] = jnp.sum(hinge, keepdims=True) - kl_x


def kernel(x, context, neg_samples, E, M_w, M_b, U_w, U_b, W_w, W_b,
           prior_mus, prior_sigmas):
    zi = jnp.zeros((), jnp.int32)
    x = x.astype(jnp.int32)
    context = context.astype(jnp.int32)
    negf = neg_samples.reshape(-1).astype(jnp.int32)
    idx_all = jnp.concatenate([
        x, jnp.full((63,), zi), context, jnp.full((142,), zi),
        negf, jnp.full((ROWS - 756,), zi)])
    idx_e = jnp.concatenate([x, jnp.full((7,), zi), context,
                             jnp.full((6,), zi)])
    sub = (idx_all & 127).reshape(ROWS, 1)

    a_arr, b_arr, mupv = _sweep(
        idx_e, E, prior_mus.T, prior_sigmas.T,
        M_w, M_b.reshape(CS, 1), U_w, U_b.reshape(CS, 1),
        W_w, W_b.reshape(CS, 1))
    ar, br = _sc_gather(idx_all, a_arr, b_arr)
    out = pl.pallas_call(
        _hinge_body,
        out_shape=jax.ShapeDtypeStruct((1, 1), jnp.float32),
    )(ar, br, sub, mupv)
    return out.reshape((1,))


# speedup vs baseline: 1.8922x; 1.0842x over previous
import functools

import jax
import jax.numpy as jnp
from jax import lax
from jax.experimental import pallas as pl
from jax.experimental.pallas import tpu as pltpu
from jax.experimental.pallas import tpu_sc as plsc

VOCAB = 100000
EMB = 128
CS = 64
CTX = 50
NEG = 10

ROWS = 1024
RPW = ROWS // 32
EROWS = 64
BLK = 10240
NBLK = 10
FROWS = NBLK * BLK // 128


def _sweep_body(idx_e_ref, e_any, pmt_ref, pst_ref, mw_ref, mbc_ref,
                uw_ref, ubc_ref, ww_ref, wbc_ref, a_ref, b_ref, mupv_ref,
                ev, sem):
    f32 = jnp.float32
    hi = jax.lax.Precision.HIGHEST
    i = pl.program_id(0)

    @pl.when(i == 0)
    def _mlp():
        def fire(j, carry):
            r = idx_e_ref[j]
            pltpu.make_async_copy(
                e_any.at[pl.ds(r, 1)], ev.at[pl.ds(j, 1)], sem).start()
            return carry
        lax.fori_loop(0, EROWS, fire, 0)
        pltpu.make_async_copy(e_any.at[pl.ds(0, EROWS)], ev, sem).wait()

        e = ev[...]
        r128 = lax.broadcasted_iota(jnp.int32, (EMB, EMB), 0)
        c128 = lax.broadcasted_iota(jnp.int32, (EMB, EMB), 1)
        ident = jnp.where(r128 == c128, 1.0, 0.0).astype(f32)
        ext = lax.dot_general(ident, e[0:8, :], (((1,), (1,)), ((), ())),
                              precision=hi, preferred_element_type=f32)
        ect = lax.dot_general(ident, e[8:64, :], (((1,), (1,)), ((), ())),
                              precision=hi, preferred_element_type=f32)
        ex_col = ext[:, 0:1]
        rw = jax.nn.relu(
            lax.dot_general(mw_ref[...], ex_col, (((1,), (0,)), ((), ())),
                            precision=hi, preferred_element_type=f32)
            + mbc_ref[...])
        rcm = jax.nn.relu(
            lax.dot_general(mw_ref[...], ect, (((1,), (0,)), ((), ())),
                            precision=hi, preferred_element_type=f32)
            + mbc_ref[...])
        ccol = lax.broadcasted_iota(jnp.int32, (CS, 56), 1)
        rcm = jnp.where(ccol < CTX, rcm, 0.0)
        h2 = jnp.sum(rcm, axis=1, keepdims=True)
        h = jnp.concatenate([CTX * rw, h2], axis=0)
        mu = lax.dot_general(uw_ref[...], h, (((1,), (0,)), ((), ())),
                             precision=hi, preferred_element_type=f32) \
            + ubc_ref[...]
        z = lax.dot_general(ww_ref[...], h, (((1,), (0,)), ((), ())),
                            precision=hi, preferred_element_type=f32) \
            + wbc_ref[...]
        pv = jax.nn.softplus(z)
        lpv = jnp.sum(jnp.log(pv))
        lane8 = lax.broadcasted_iota(jnp.int32, (CS, 8), 1)
        mupv_ref[...] = (
            jnp.where(lane8 == 0, jnp.broadcast_to(mu, (CS, 8)), 0.0)
            + jnp.where(lane8 == 1, jnp.broadcast_to(pv, (CS, 8)), 0.0)
            + jnp.where(lane8 == 2, lpv, 0.0))

    mu_col = mupv_ref[:, 0:1]
    pv_col = mupv_ref[:, 1:2]
    pm = pmt_ref[...]
    s = pst_ref[...]
    v = s * s
    d = pm - mu_col
    a_part = jnp.sum((pv_col + d * d) / v, axis=0, keepdims=True)
    p = s
    for half in (32, 16, 8, 4, 2, 1):
        p = p[0:half, :] * p[half:2 * half, :]
    b_part = 2.0 * jnp.log(p)
    a_ref[...] = jnp.concatenate(
        [a_part[:, k * 128:(k + 1) * 128] for k in range(BLK // 128)], axis=0)
    b_ref[...] = jnp.concatenate(
        [b_part[:, k * 128:(k + 1) * 128] for k in range(BLK // 128)], axis=0)


def _sweep(idx_e, E, pmT, psT, M_w, M_bc, U_w, U_bc, W_w, W_bc):
    last = VOCAB // BLK
    return pl.pallas_call(
        _sweep_body,
        grid=(NBLK,),
        in_specs=[
            pl.BlockSpec(memory_space=pltpu.SMEM),
            pl.BlockSpec(memory_space=pl.ANY),
            pl.BlockSpec((CS, BLK), lambda i: (0, jnp.minimum(i, last))),
            pl.BlockSpec((CS, BLK), lambda i: (0, jnp.minimum(i, last))),
            pl.BlockSpec((CS, EMB), lambda i: (0, 0)),
            pl.BlockSpec((CS, 1), lambda i: (0, 0)),
            pl.BlockSpec((CS, EMB), lambda i: (0, 0)),
            pl.BlockSpec((CS, 1), lambda i: (0, 0)),
            pl.BlockSpec((CS, EMB), lambda i: (0, 0)),
            pl.BlockSpec((CS, 1), lambda i: (0, 0)),
        ],
        out_specs=[
            pl.BlockSpec((BLK // 128, 128), lambda i: (i, 0)),
            pl.BlockSpec((BLK // 128, 128), lambda i: (i, 0)),
            pl.BlockSpec((CS, 8), lambda i: (0, 0)),
        ],
        out_shape=[
            jax.ShapeDtypeStruct((FROWS, 128), jnp.float32),
            jax.ShapeDtypeStruct((FROWS, 128), jnp.float32),
            jax.ShapeDtypeStruct((CS, 8), jnp.float32),
        ],
        scratch_shapes=[pltpu.VMEM((EROWS, EMB), jnp.float32),
                        pltpu.SemaphoreType.DMA],
    )(idx_e, E, pmT, psT, M_w, M_bc, U_w, U_bc, W_w, W_bc)


def _sc_gather(idx_all, a_arr, b_arr):
    mesh = plsc.VectorSubcoreMesh(core_axis_name="c", subcore_axis_name="s")

    @functools.partial(
        pl.kernel,
        out_type=(
            jax.ShapeDtypeStruct((ROWS, 128), jnp.float32),
            jax.ShapeDtypeStruct((ROWS, 128), jnp.float32),
        ),
        mesh=mesh,
        scratch_types=(
            pltpu.VMEM((RPW,), jnp.int32),
            pltpu.VMEM((RPW,), jnp.int32),
            pltpu.VMEM((RPW, 128), jnp.float32),
            pltpu.VMEM((RPW, 128), jnp.float32),
            pltpu.SemaphoreType.DMA,
            pltpu.SemaphoreType.DMA,
        ),
    )
    def k(idx_hbm, a_hbm, b_hbm, out_a, out_b, idxv, tilev, rows_a, rows_b,
          sem0, sem1):
        wid = lax.axis_index("s") * 2 + lax.axis_index("c")
        base = wid * RPW
        pltpu.sync_copy(idx_hbm.at[pl.ds(base, RPW)], idxv)
        for g in range(RPW // 16):
            iv = idxv[pl.ds(16 * g, 16)]
            tilev[pl.ds(16 * g, 16)] = lax.shift_right_logical(iv, 7)
        cp0 = pltpu.async_copy(a_hbm.at[tilev], rows_a, sem0)
        cp1 = pltpu.async_copy(b_hbm.at[tilev], rows_b, sem1)
        cp0.wait()
        pltpu.sync_copy(rows_a, out_a.at[pl.ds(base, RPW)])
        cp1.wait()
        pltpu.sync_copy(rows_b, out_b.at[pl.ds(base, RPW)])

    return k(idx_all, a_arr, b_arr)


def _hinge_body(ar_ref, br_ref, sub_ref, mupv_ref, out_ref):
    f32 = jnp.float32
    hi = jax.lax.Precision.HIGHEST
    lpv = mupv_ref[0:1, 2:3]
    lane = lax.broadcasted_iota(jnp.int32, (ROWS, 128), 1)
    onehot = jnp.where(lane == sub_ref[...], 1.0, 0.0).astype(f32)
    a = jnp.sum(ar_ref[...] * onehot, axis=1, keepdims=True)
    b = jnp.sum(br_ref[...] * onehot, axis=1, keepdims=True)
    kl = 0.5 * (a + b - CS - lpv)

    kl_x = kl[0:1, 0:1]
    kl_pos = kl[64:128, :]
    kl_neg = kl[256:768, :]
    irow = lax.broadcasted_iota(jnp.int32, (512, 64), 0)
    icol = lax.broadcasted_iota(jnp.int32, (512, 64), 1)
    sel = jnp.where(irow // NEG == icol, 1.0, 0.0).astype(f32)
    pos_for_neg = lax.dot_general(sel, kl_pos, (((1,), (0,)), ((), ())),
                                  precision=hi, preferred_element_type=f32)
    hinge = jnp.maximum(kl_neg - pos_for_neg + 1.0, 0.0)
    nrow = lax.broadcasted_iota(jnp.int32, (512, 1), 0)
    hinge = jnp.where(nrow < CTX * NEG, hinge, 0.0)
    out_ref[...] = jnp.sum(hinge, keepdims=True) - kl_x


def kernel(x, context, neg_samples, E, M_w, M_b, U_w, U_b, W_w, W_b,
           prior_mus, prior_sigmas):
    zi = jnp.zeros((), jnp.int32)
    x = x.astype(jnp.int32)
    context = context.astype(jnp.int32)
    negf = neg_samples.reshape(-1).astype(jnp.int32)
    idx_all = jnp.concatenate([
        x, jnp.full((63,), zi), context, jnp.full((142,), zi),
        negf, jnp.full((ROWS - 756,), zi)])
    idx_e = jnp.concatenate([x, jnp.full((7,), zi), context,
                             jnp.full((6,), zi)])
    sub = (idx_all & 127).reshape(ROWS, 1)

    a_arr, b_arr, mupv = _sweep(
        idx_e, E, prior_mus.T, prior_sigmas.T,
        M_w, M_b.reshape(CS, 1), U_w, U_b.reshape(CS, 1),
        W_w, W_b.reshape(CS, 1))
    ar, br = _sc_gather(idx_all, a_arr, b_arr)
    out = pl.pallas_call(
        _hinge_body,
        out_shape=jax.ShapeDtypeStruct((1, 1), jnp.float32),
    )(ar, br, sub, mupv)
    return out.reshape((1,))
